# baseline (speedup 1.0000x reference)
import functools

import jax
import jax.numpy as jnp
from jax import lax
from jax.experimental import pallas as pl
from jax.experimental.pallas import tpu as pltpu
from jax.experimental.pallas import tpu_sc as plsc

NUM_EMBEDDINGS = 1000000
EMBED_DIM = 32
BATCH = 4096
SEQ = 200

N = BATCH * SEQ
NC, NS = 2, 16
NW = NC * NS
CH = 512
NB_E = BATCH // CH
UNITS = SEQ * NB_E
PER_W = UNITS // NW
TILE_I = CH // 128


@functools.partial(
    pl.kernel,
    out_type=jax.ShapeDtypeStruct((SEQ, 4, BATCH // 128, 8, 128), jnp.float32),
    mesh=plsc.VectorSubcoreMesh(core_axis_name="c", subcore_axis_name="s"),
    compiler_params=pltpu.CompilerParams(
        use_tc_tiling_on_sc=False, needs_layout_passes=False),
    scratch_types=[
        pltpu.VMEM((CH,), jnp.int32),
        pltpu.VMEM((CH,), jnp.int32),
        pltpu.VMEM((CH, EMBED_DIM), jnp.float32),
        pltpu.VMEM((CH, EMBED_DIM), jnp.float32),
        pltpu.VMEM((4, TILE_I, 8, 128), jnp.float32),
        pltpu.VMEM((4, TILE_I, 8, 128), jnp.float32),
        pltpu.SemaphoreType.DMA,
        pltpu.SemaphoreType.DMA,
        pltpu.SemaphoreType.DMA,
        pltpu.SemaphoreType.DMA,
    ],
)
def _gather_kernel(idx_hbm, table_hbm, out_hbm,
                   idx0, idx1, rows0, rows1, dst0, dst1,
                   sg0, sg1, so0, so1):
    wid = lax.axis_index("s") * NC + lax.axis_index("c")
    base_u = wid * PER_W
    idx_v = (idx0, idx1)
    rows = (rows0, rows1)
    dst = (dst0, dst1)
    sg = (sg0, sg1)
    so = (so0, so1)
    iota16 = lax.iota(jnp.int32, 16)

    def chunk_se(c):
        u = base_u + c
        s = u // NB_E
        e = u - s * NB_E
        return s, e

    def fire_gather(c, b):
        s, e = chunk_se(c)
        pltpu.sync_copy(idx_hbm.at[pl.ds(s * BATCH + e * CH, CH)], idx_v[b])
        pltpu.async_copy(table_hbm.at[idx_v[b]], rows[b], sg[b])

    def wait_gather(b):
        pltpu.make_async_copy(table_hbm.at[idx_v[b]], rows[b], sg[b]).wait()

    def out_slice(c):
        s, e = chunk_se(c)
        return out_hbm.at[s, :, pl.ds(e * TILE_I, TILE_I), :, :]

    def fire_store(c, b):
        pltpu.async_copy(dst[b], out_slice(c), so[b])

    def wait_store(c_now, b):
        pltpu.make_async_copy(dst[b], out_slice(c_now), so[b]).wait()

    def transpose_chunk(b):
        rows_v = rows[b]
        dst_v = dst[b]

        def tbody(t, carry):
            jd = t // TILE_I
            id_ = t - jd * TILE_I
            for jr in range(8):
                colv = jnp.full((16,), jd * 8 + jr, jnp.int32)
                for k in range(8):
                    rowv = iota16 + (id_ * 128 + k * 16)
                    vec = plsc.load_gather(rows_v, [rowv, colv])
                    dst_v[jd, id_, jr, pl.ds(k * 16, 16)] = vec
            return carry

        lax.fori_loop(0, 4 * TILE_I, tbody, 0)

    fire_gather(0, 0)

    def body(i, carry):
        ca = 2 * i
        cb = 2 * i + 1
        fire_gather(cb, 1)
        wait_gather(0)

        @pl.when(i > 0)
        def _():
            wait_store(ca, 0)

        transpose_chunk(0)
        fire_store(ca, 0)

        @pl.when(i < PER_W // 2 - 1)
        def _():
            fire_gather(ca + 2, 0)

        wait_gather(1)

        @pl.when(i > 0)
        def _():
            wait_store(cb, 1)

        transpose_chunk(1)
        fire_store(cb, 1)
        return carry

    lax.fori_loop(0, PER_W // 2, body, 0)

    wait_store(PER_W - 2, 0)
    wait_store(PER_W - 1, 1)


def kernel(indices, table):
    idx_s_major = indices.T.reshape(N)
    out5 = _gather_kernel(idx_s_major, table)
    return out5.transpose((2, 4, 0, 1, 3)).reshape(BATCH, SEQ, EMBED_DIM)


# speedup vs baseline: 1.4948x; 1.4948x over previous
import functools

import jax
import jax.numpy as jnp
from jax import lax
from jax.experimental import pallas as pl
from jax.experimental.pallas import tpu as pltpu
from jax.experimental.pallas import tpu_sc as plsc

NUM_EMBEDDINGS = 1000000
EMBED_DIM = 32
BATCH = 4096
SEQ = 200

N = BATCH * SEQ
NC, NS = 2, 16
NW = NC * NS
CH = 512
NB_E = BATCH // CH
UNITS = SEQ * NB_E
PER_W = UNITS // NW
TILE_I = CH // 128


@functools.partial(
    pl.kernel,
    out_type=jax.ShapeDtypeStruct((SEQ, 4, BATCH // 128, 8, 128), jnp.float32),
    mesh=plsc.VectorSubcoreMesh(core_axis_name="c", subcore_axis_name="s"),
    compiler_params=pltpu.CompilerParams(
        use_tc_tiling_on_sc=False, needs_layout_passes=False),
    scratch_types=[
        pltpu.VMEM((CH,), jnp.int32),
        pltpu.VMEM((CH,), jnp.int32),
        pltpu.VMEM((CH, EMBED_DIM), jnp.float32),
        pltpu.VMEM((CH, EMBED_DIM), jnp.float32),
        pltpu.VMEM((4, TILE_I, 8, 128), jnp.float32),
        pltpu.VMEM((4, TILE_I, 8, 128), jnp.float32),
        pltpu.SemaphoreType.DMA,
        pltpu.SemaphoreType.DMA,
        pltpu.SemaphoreType.DMA,
        pltpu.SemaphoreType.DMA,
        pltpu.SemaphoreType.DMA,
        pltpu.SemaphoreType.DMA,
    ],
)
def _gather_kernel(idx_hbm, table_hbm, out_hbm,
                   idx0, idx1, rows0, rows1, dst0, dst1,
                   si0, si1, sg0, sg1, so0, so1):
    wid = lax.axis_index("s") * NC + lax.axis_index("c")
    base_u = wid * PER_W
    idx_v = (idx0, idx1)
    rows = (rows0, rows1)
    dst = (dst0, dst1)
    si = (si0, si1)
    sg = (sg0, sg1)
    so = (so0, so1)
    iota16 = lax.iota(jnp.int32, 16)
    cp8 = [jnp.bitwise_and(iota16 + r, 7) for r in range(8)]
    jv_const = [jnp.full((16,), j, jnp.int32) for j in range(4)]

    def chunk_se(c):
        u = base_u + c
        s = u // NB_E
        e = u - s * NB_E
        return s, e

    def fire_gather(c, b):
        s, e = chunk_se(c)
        sb = s // 8
        sr = s - sb * 8
        for r in range(TILE_I):
            pltpu.async_copy(idx_hbm.at[sb, e * TILE_I + r, sr, :],
                             idx_v[b].at[pl.ds(r * 128, 128)], si[b])
        for r in range(TILE_I):
            pltpu.make_async_copy(idx_hbm.at[sb, e * TILE_I + r, sr, :],
                                  idx_v[b].at[pl.ds(r * 128, 128)],
                                  si[b]).wait()
        pltpu.async_copy(table_hbm.at[idx_v[b]], rows[b], sg[b])

    def wait_gather(b):
        pltpu.make_async_copy(table_hbm.at[idx_v[b]], rows[b], sg[b]).wait()

    def out_slice(c):
        s, e = chunk_se(c)
        return out_hbm.at[s, :, pl.ds(e * TILE_I, TILE_I), :, :]

    def fire_store(c, b):
        pltpu.async_copy(dst[b], out_slice(c), so[b])

    def wait_store(c_now, b):
        pltpu.make_async_copy(dst[b], out_slice(c_now), so[b]).wait()

    def transpose_chunk(b):
        rows_v = rows[b]
        dst_v = dst[b]

        def tbody(bb, carry):
            il = bb // 8
            bc0 = (bb - il * 8) * 16
            rowv = iota16 + bb * 16
            bcv = iota16 + bc0
            ilv = jnp.full((16,), il, jnp.int32)
            for j in range(4):
                for r in range(8):
                    colv = cp8[r] + j * 8
                    vec = plsc.load_gather(rows_v, [rowv, colv])
                    plsc.store_scatter(dst_v, [jv_const[j], ilv, cp8[r], bcv],
                                      vec)
            return carry

        lax.fori_loop(0, CH // 16, tbody, 0)

    fire_gather(0, 0)

    def body(i, carry):
        ca = 2 * i
        cb = 2 * i + 1
        fire_gather(cb, 1)
        wait_gather(0)

        @pl.when(i > 0)
        def _():
            wait_store(ca, 0)

        transpose_chunk(0)
        fire_store(ca, 0)

        @pl.when(i < PER_W // 2 - 1)
        def _():
            fire_gather(ca + 2, 0)

        wait_gather(1)

        @pl.when(i > 0)
        def _():
            wait_store(cb, 1)

        transpose_chunk(1)
        fire_store(cb, 1)
        return carry

    lax.fori_loop(0, PER_W // 2, body, 0)

    wait_store(PER_W - 2, 0)
    wait_store(PER_W - 1, 1)


def kernel(indices, table):
    idx4 = indices.T.reshape(SEQ // 8, 8, BATCH // 128, 128).transpose(0, 2, 1, 3)
    out5 = _gather_kernel(idx4, table)
    return out5.transpose((2, 4, 0, 1, 3)).reshape(BATCH, SEQ, EMBED_DIM)


# speedup vs baseline: 1.7480x; 1.1694x over previous
import functools

import jax
import jax.numpy as jnp
from jax import lax
from jax.experimental import pallas as pl
from jax.experimental.pallas import tpu as pltpu
from jax.experimental.pallas import tpu_sc as plsc

NUM_EMBEDDINGS = 1000000
EMBED_DIM = 32
BATCH = 4096
SEQ = 200

N = BATCH * SEQ
NC, NS = 2, 16
NW = NC * NS
CH = 512
NB_E = BATCH // CH
UNITS = SEQ * NB_E
PER_W = UNITS // NW
TILE_I = CH // 128


@functools.partial(
    pl.kernel,
    out_type=jax.ShapeDtypeStruct((SEQ, 4, BATCH // 128, 8, 128), jnp.float32),
    mesh=plsc.VectorSubcoreMesh(core_axis_name="c", subcore_axis_name="s"),
    compiler_params=pltpu.CompilerParams(use_tc_tiling_on_sc=False),
    scratch_types=[
        pltpu.VMEM((CH,), jnp.int32),
        pltpu.VMEM((CH,), jnp.int32),
        pltpu.VMEM((CH, EMBED_DIM), jnp.float32),
        pltpu.VMEM((CH, EMBED_DIM), jnp.float32),
        pltpu.VMEM((4, TILE_I, 8, 128), jnp.float32),
        pltpu.VMEM((4, TILE_I, 8, 128), jnp.float32),
        pltpu.SemaphoreType.DMA,
        pltpu.SemaphoreType.DMA,
        pltpu.SemaphoreType.DMA,
        pltpu.SemaphoreType.DMA,
        pltpu.SemaphoreType.DMA,
        pltpu.SemaphoreType.DMA,
    ],
)
def _gather_kernel(idx_hbm, table_hbm, out_hbm,
                   idx0, idx1, rows0, rows1, dst0, dst1,
                   si0, si1, sg0, sg1, so0, so1):
    wid = lax.axis_index("s") * NC + lax.axis_index("c")
    base_u = wid * PER_W
    idx_v = (idx0, idx1)
    rows = (rows0, rows1)
    dst = (dst0, dst1)
    si = (si0, si1)
    sg = (sg0, sg1)
    so = (so0, so1)
    iota16 = lax.iota(jnp.int32, 16)
    xor_perm = [iota16 ^ k for k in (1, 2, 4, 8)]
    lane_bit = [jnp.bitwise_and(iota16, k) for k in (1, 2, 4, 8)]

    def chunk_se(c):
        u = base_u + c
        s = u // NB_E
        e = u - s * NB_E
        return s, e

    def fire_gather(c, b):
        s, e = chunk_se(c)
        sb = s // 8
        sr = s - sb * 8
        for r in range(TILE_I):
            pltpu.async_copy(idx_hbm.at[sb, e * TILE_I + r, sr, :],
                             idx_v[b].at[pl.ds(r * 128, 128)], si[b])
        for r in range(TILE_I):
            pltpu.make_async_copy(idx_hbm.at[sb, e * TILE_I + r, sr, :],
                                  idx_v[b].at[pl.ds(r * 128, 128)],
                                  si[b]).wait()
        pltpu.async_copy(table_hbm.at[idx_v[b]], rows[b], sg[b])

    def wait_gather(b):
        pltpu.make_async_copy(table_hbm.at[idx_v[b]], rows[b], sg[b]).wait()

    def out_slice(c):
        s, e = chunk_se(c)
        return out_hbm.at[s, :, pl.ds(e * TILE_I, TILE_I), :, :]

    def fire_store(c, b):
        pltpu.async_copy(dst[b], out_slice(c), so[b])

    def wait_store(c_now, b):
        pltpu.make_async_copy(dst[b], out_slice(c_now), so[b]).wait()

    def transpose_chunk(b):
        rows_v = rows[b]
        dst_v = dst[b]

        def tbody(bb, carry):
            il = bb // 8
            bc0 = (bb - il * 8) * 16
            b0 = bb * 16
            for j0 in (0, 16):
                v = [rows_v[b0 + i, pl.ds(j0, 16)] for i in range(16)]
                for st, k in enumerate((1, 2, 4, 8)):
                    keep = [lane_bit[st] == (i & k) for i in range(16)]
                    sw = [v[i].at[xor_perm[st]].get(mode="promise_in_bounds")
                          for i in range(16)]
                    v = [jnp.where(keep[i], v[i], sw[i ^ k])
                         for i in range(16)]
                for jj in range(16):
                    j = j0 + jj
                    dst_v[j // 8, il, j % 8, pl.ds(bc0, 16)] = v[jj]
            return carry

        lax.fori_loop(0, CH // 16, tbody, 0)

    fire_gather(0, 0)

    def body(i, carry):
        ca = 2 * i
        cb = 2 * i + 1
        fire_gather(cb, 1)
        wait_gather(0)

        @pl.when(i > 0)
        def _():
            wait_store(ca, 0)

        transpose_chunk(0)
        fire_store(ca, 0)

        @pl.when(i < PER_W // 2 - 1)
        def _():
            fire_gather(ca + 2, 0)

        wait_gather(1)

        @pl.when(i > 0)
        def _():
            wait_store(cb, 1)

        transpose_chunk(1)
        fire_store(cb, 1)
        return carry

    lax.fori_loop(0, PER_W // 2, body, 0)

    wait_store(PER_W - 2, 0)
    wait_store(PER_W - 1, 1)


def kernel(indices, table):
    idx4 = indices.T.reshape(SEQ // 8, 8, BATCH // 128, 128).transpose(0, 2, 1, 3)
    out5 = _gather_kernel(idx4, table)
    return out5.transpose((2, 4, 0, 1, 3)).reshape(BATCH, SEQ, EMBED_DIM)


# speedup vs baseline: 1.8445x; 1.0552x over previous
import functools

import jax
import jax.numpy as jnp
from jax import lax
from jax.experimental import pallas as pl
from jax.experimental.pallas import tpu as pltpu
from jax.experimental.pallas import tpu_sc as plsc

NUM_EMBEDDINGS = 1000000
EMBED_DIM = 32
BATCH = 4096
SEQ = 200

N = BATCH * SEQ
NC, NS = 2, 16
NW = NC * NS
CH = 512
NB_E = BATCH // CH
UNITS = SEQ * NB_E
PER_W = UNITS // NW
TILE_I = CH // 128


@functools.partial(
    pl.kernel,
    out_type=jax.ShapeDtypeStruct((SEQ, 4, BATCH // 128, 8, 128), jnp.float32),
    mesh=plsc.VectorSubcoreMesh(core_axis_name="c", subcore_axis_name="s"),
    compiler_params=pltpu.CompilerParams(use_tc_tiling_on_sc=False),
    scratch_types=[
        pltpu.VMEM((CH,), jnp.int32),
        pltpu.VMEM((CH,), jnp.int32),
        pltpu.VMEM((CH, EMBED_DIM), jnp.float32),
        pltpu.VMEM((CH, EMBED_DIM), jnp.float32),
        pltpu.VMEM((4, TILE_I, 8, 128), jnp.float32),
        pltpu.VMEM((4, TILE_I, 8, 128), jnp.float32),
        pltpu.SemaphoreType.DMA,
        pltpu.SemaphoreType.DMA,
        pltpu.SemaphoreType.DMA,
        pltpu.SemaphoreType.DMA,
        pltpu.SemaphoreType.DMA,
        pltpu.SemaphoreType.DMA,
    ],
)
def _gather_kernel(idx_hbm, table_hbm, out_hbm,
                   idx0, idx1, rows0, rows1, dst0, dst1,
                   si0, si1, sg0, sg1, so0, so1):
    wid = lax.axis_index("s") * NC + lax.axis_index("c")
    base_u = wid * PER_W
    idx_v = (idx0, idx1)
    rows = (rows0, rows1)
    dst = (dst0, dst1)
    si = (si0, si1)
    sg = (sg0, sg1)
    so = (so0, so1)
    iota16 = lax.iota(jnp.int32, 16)
    xor_perm = [iota16 ^ k for k in (1, 2, 4, 8)]
    lane_bit = [jnp.bitwise_and(iota16, k) for k in (1, 2, 4, 8)]

    def chunk_se(c):
        u = base_u + c
        s = u // NB_E
        e = u - s * NB_E
        return s, e

    def fire_idx(c, b):
        s, e = chunk_se(c)
        sb = s // 8
        sr = s - sb * 8
        for r in range(TILE_I):
            pltpu.async_copy(idx_hbm.at[sb, e * TILE_I + r, sr, :],
                             idx_v[b].at[pl.ds(r * 128, 128)], si[b])

    def wait_idx(c, b):
        s, e = chunk_se(c)
        sb = s // 8
        sr = s - sb * 8
        for r in range(TILE_I):
            pltpu.make_async_copy(idx_hbm.at[sb, e * TILE_I + r, sr, :],
                                  idx_v[b].at[pl.ds(r * 128, 128)],
                                  si[b]).wait()

    def fire_gather(c, b):
        pltpu.async_copy(table_hbm.at[idx_v[b]], rows[b], sg[b])

    def wait_gather(b):
        pltpu.make_async_copy(table_hbm.at[idx_v[b]], rows[b], sg[b]).wait()

    def out_slice(c):
        s, e = chunk_se(c)
        return out_hbm.at[s, :, pl.ds(e * TILE_I, TILE_I), :, :]

    def fire_store(c, b):
        pltpu.async_copy(dst[b], out_slice(c), so[b])

    def wait_store(c_now, b):
        pltpu.make_async_copy(dst[b], out_slice(c_now), so[b]).wait()

    def transpose_chunk(b):
        rows_v = rows[b]
        dst_v = dst[b]

        def tbody(bb, carry):
            il = bb // 8
            bc0 = (bb - il * 8) * 16
            b0 = bb * 16
            for j0 in (0, 16):
                v = [rows_v[b0 + i, pl.ds(j0, 16)] for i in range(16)]
                for st, k in enumerate((1, 2, 4, 8)):
                    keep = [lane_bit[st] == (i & k) for i in range(16)]
                    sw = [v[i].at[xor_perm[st]].get(mode="promise_in_bounds")
                          for i in range(16)]
                    v = [jnp.where(keep[i], v[i], sw[i ^ k])
                         for i in range(16)]
                for jj in range(16):
                    j = j0 + jj
                    dst_v[j // 8, il, j % 8, pl.ds(bc0, 16)] = v[jj]
            return carry

        lax.fori_loop(0, CH // 16, tbody, 0)

    fire_idx(0, 0)
    fire_idx(1, 1)
    wait_idx(0, 0)
    fire_gather(0, 0)

    def body(i, carry):
        ca = 2 * i
        cb = 2 * i + 1
        wait_idx(cb, 1)
        fire_gather(cb, 1)
        wait_gather(0)

        @pl.when(i > 0)
        def _():
            wait_store(ca, 0)

        @pl.when(i < PER_W // 2 - 1)
        def _():
            fire_idx(ca + 2, 0)

        transpose_chunk(0)
        fire_store(ca, 0)

        @pl.when(i < PER_W // 2 - 1)
        def _():
            wait_idx(ca + 2, 0)
            fire_gather(ca + 2, 0)

        wait_gather(1)

        @pl.when(i > 0)
        def _():
            wait_store(cb, 1)

        @pl.when(i < PER_W // 2 - 1)
        def _():
            fire_idx(cb + 2, 1)

        transpose_chunk(1)
        fire_store(cb, 1)
        return carry

    lax.fori_loop(0, PER_W // 2, body, 0)

    wait_store(PER_W - 2, 0)
    wait_store(PER_W - 1, 1)


def kernel(indices, table):
    idx4 = indices.T.reshape(SEQ // 8, 8, BATCH // 128, 128).transpose(0, 2, 1, 3)
    out5 = _gather_kernel(idx4, table)
    return out5.transpose((2, 4, 0, 1, 3)).reshape(BATCH, SEQ, EMBED_DIM)


# speedup vs baseline: 1.8587x; 1.0077x over previous
import functools

import jax
import jax.numpy as jnp
from jax import lax
from jax.experimental import pallas as pl
from jax.experimental.pallas import tpu as pltpu
from jax.experimental.pallas import tpu_sc as plsc

NUM_EMBEDDINGS = 1000000
EMBED_DIM = 32
BATCH = 4096
SEQ = 200

N = BATCH * SEQ
NC, NS = 2, 16
NW = NC * NS
CH = 512
NB_E = BATCH // CH
UNITS = SEQ * NB_E
PER_W = UNITS // NW
TILE_I = CH // 128


@functools.partial(
    pl.kernel,
    out_type=jax.ShapeDtypeStruct((SEQ, 4, BATCH // 128, 8, 128), jnp.float32),
    mesh=plsc.VectorSubcoreMesh(core_axis_name="c", subcore_axis_name="s"),
    compiler_params=pltpu.CompilerParams(use_tc_tiling_on_sc=False),
    scratch_types=[
        pltpu.VMEM((CH,), jnp.int32),
        pltpu.VMEM((CH,), jnp.int32),
        pltpu.VMEM((CH, EMBED_DIM), jnp.float32),
        pltpu.VMEM((CH, EMBED_DIM), jnp.float32),
        pltpu.VMEM((4, TILE_I, 8, 128), jnp.float32),
        pltpu.VMEM((4, TILE_I, 8, 128), jnp.float32),
        pltpu.SemaphoreType.DMA,
        pltpu.SemaphoreType.DMA,
        pltpu.SemaphoreType.DMA,
        pltpu.SemaphoreType.DMA,
        pltpu.SemaphoreType.DMA,
        pltpu.SemaphoreType.DMA,
    ],
)
def _gather_kernel(idx_hbm, table_hbm, out_hbm,
                   idx0, idx1, rows0, rows1, dst0, dst1,
                   si0, si1, sg0, sg1, so0, so1):
    wid = lax.axis_index("s") * NC + lax.axis_index("c")
    base_u = wid * PER_W
    idx_v = (idx0, idx1)
    rows = (rows0, rows1)
    dst = (dst0, dst1)
    si = (si0, si1)
    sg = (sg0, sg1)
    so = (so0, so1)
    iota16 = lax.iota(jnp.int32, 16)
    xor_perm = [iota16 ^ k for k in (1, 2, 4, 8)]
    lane_bit = [jnp.bitwise_and(iota16, k) for k in (1, 2, 4, 8)]

    def chunk_se(c):
        u = base_u + c
        s = u // NB_E
        e = u - s * NB_E
        return s, e

    def fire_idx(c, b):
        s, e = chunk_se(c)
        sb = s // 8
        sr = s - sb * 8
        for r in range(TILE_I):
            pltpu.async_copy(idx_hbm.at[sb, e * TILE_I + r, sr, :],
                             idx_v[b].at[pl.ds(r * 128, 128)], si[b])

    def wait_idx(c, b):
        s, e = chunk_se(c)
        sb = s // 8
        sr = s - sb * 8
        for r in range(TILE_I):
            pltpu.make_async_copy(idx_hbm.at[sb, e * TILE_I + r, sr, :],
                                  idx_v[b].at[pl.ds(r * 128, 128)],
                                  si[b]).wait()
        def sbody(k, carry):
            off = k * 16
            idx_v[b][pl.ds(off, 16)] = idx_v[b][pl.ds(off, 16)] * 4
            return carry
        lax.fori_loop(0, CH // 16, sbody, 0)

    def fire_gather(c, b):
        pltpu.async_copy(table_hbm.at[idx_v[b]], rows[b], sg[b])

    def wait_gather(b):
        pltpu.make_async_copy(table_hbm.at[idx_v[b]], rows[b], sg[b]).wait()

    def out_slice(c):
        s, e = chunk_se(c)
        return out_hbm.at[s, :, pl.ds(e * TILE_I, TILE_I), :, :]

    def fire_store(c, b):
        pltpu.async_copy(dst[b], out_slice(c), so[b])

    def wait_store(c_now, b):
        pltpu.make_async_copy(dst[b], out_slice(c_now), so[b]).wait()

    def transpose_chunk(b):
        rows_v = rows[b]
        dst_v = dst[b]

        def tbody(bb, carry):
            il = bb // 8
            bc0 = (bb - il * 8) * 16
            b0 = bb * 16
            for j0 in (0, 16):
                v = [rows_v[b0 + i, pl.ds(j0, 16)] for i in range(16)]
                for st, k in enumerate((1, 2, 4, 8)):
                    keep = [lane_bit[st] == (i & k) for i in range(16)]
                    sw = [v[i].at[xor_perm[st]].get(mode="promise_in_bounds")
                          for i in range(16)]
                    v = [jnp.where(keep[i], v[i], sw[i ^ k])
                         for i in range(16)]
                for jj in range(16):
                    j = j0 + jj
                    dst_v[j // 8, il, j % 8, pl.ds(bc0, 16)] = v[jj]
            return carry

        lax.fori_loop(0, CH // 16, tbody, 0)

    fire_idx(0, 0)
    fire_idx(1, 1)
    wait_idx(0, 0)
    fire_gather(0, 0)

    def body(i, carry):
        ca = 2 * i
        cb = 2 * i + 1
        wait_idx(cb, 1)
        fire_gather(cb, 1)
        wait_gather(0)

        @pl.when(i > 0)
        def _():
            wait_store(ca, 0)

        @pl.when(i < PER_W // 2 - 1)
        def _():
            fire_idx(ca + 2, 0)

        transpose_chunk(0)
        fire_store(ca, 0)

        @pl.when(i < PER_W // 2 - 1)
        def _():
            wait_idx(ca + 2, 0)
            fire_gather(ca + 2, 0)

        wait_gather(1)

        @pl.when(i > 0)
        def _():
            wait_store(cb, 1)

        @pl.when(i < PER_W // 2 - 1)
        def _():
            fire_idx(cb + 2, 1)

        transpose_chunk(1)
        fire_store(cb, 1)
        return carry

    lax.fori_loop(0, PER_W // 2, body, 0)

    wait_store(PER_W - 2, 0)
    wait_store(PER_W - 1, 1)


def kernel(indices, table):
    idx4 = indices.T.reshape(SEQ // 8, 8, BATCH // 128, 128).transpose(0, 2, 1, 3)
    table_p = lax.pad(table, jnp.float32(0), ((0, 0, 0), (0, 96, 0)))
    out5 = _gather_kernel(idx4, table_p.reshape(4 * NUM_EMBEDDINGS, EMBED_DIM))
    return out5.transpose((2, 4, 0, 1, 3)).reshape(BATCH, SEQ, EMBED_DIM)
